# Initial kernel scaffold; baseline (speedup 1.0000x reference)
import jax
import jax.numpy as jnp
from jax.experimental import pallas as pl


def kernel(src_seq, structure_emb):
    raise NotImplementedError("write your pallas kernel here")


# speedup vs baseline: 2.6808x; 2.6808x over previous
import jax
import jax.numpy as jnp
from jax import lax
from jax.experimental import pallas as pl
from jax.experimental.pallas import tpu as pltpu
from jax.experimental.pallas import tpu_sc as plsc

N_HEADS = 12
D_EMB = 64
CHUNK = 80


def _sc_body(idx_hbm, table_hbm, out_hbm, idx_v, rows_v, gsem, wsem):
    n_rows = idx_hbm.shape[0]
    n_chunks = n_rows // CHUNK
    rows_per_batch = n_rows // 2
    chunks_per_batch = rows_per_batch // CHUNK

    info = plsc.get_sparse_core_info()
    nc, ns = info.num_cores, info.num_subcores
    n_workers = nc * ns
    wid = lax.axis_index("s") * nc + lax.axis_index("c")
    n_iter = (n_chunks + n_workers - 1) // n_workers

    def body(i, carry):
        c = wid + i * n_workers

        @pl.when(c < n_chunks)
        def _():
            pltpu.sync_copy(idx_hbm.at[pl.ds(c * CHUNK, CHUNK)], idx_v)
            pltpu.async_copy(table_hbm.at[idx_v], rows_v, gsem).wait()
            b = c // chunks_per_batch
            pos = (c % chunks_per_batch) * CHUNK
            handles = []
            for k in range(N_HEADS):
                g = 2 * k + b
                base = g * rows_per_batch + pos
                handles.append(
                    pltpu.async_copy(rows_v, out_hbm.at[pl.ds(base, CHUNK)], wsem))
            for hd in handles:
                hd.wait()

        return carry

    lax.fori_loop(0, n_iter, body, 0)


def kernel(src_seq, structure_emb):
    batch, num_posts, _ = src_seq.shape
    idx = src_seq.reshape(-1).astype(jnp.int32)
    table = structure_emb.astype(jnp.float32)
    n_rows_out = batch * N_HEADS * num_posts * num_posts

    mesh = plsc.VectorSubcoreMesh(core_axis_name="c", subcore_axis_name="s")
    f = pl.kernel(
        _sc_body,
        out_type=jax.ShapeDtypeStruct((n_rows_out, D_EMB), jnp.float32),
        mesh=mesh,
        scratch_types=[
            pltpu.VMEM((CHUNK,), jnp.int32),
            pltpu.VMEM((CHUNK, D_EMB), jnp.float32),
            pltpu.SemaphoreType.DMA,
            pltpu.SemaphoreType.DMA,
        ],
        compiler_params=pltpu.CompilerParams(use_tc_tiling_on_sc=False),
    )
    out = f(idx, table)
    return out.reshape(batch, N_HEADS, num_posts, num_posts, D_EMB)


# speedup vs baseline: 2.7658x; 1.0317x over previous
import jax
import jax.numpy as jnp
from jax import lax
from jax.experimental import pallas as pl
from jax.experimental.pallas import tpu as pltpu
from jax.experimental.pallas import tpu_sc as plsc

N_HEADS = 12
D_EMB = 64
UNIT = 1000
GSUB = 120
N_FULL = UNIT // GSUB
TAIL = UNIT - N_FULL * GSUB


def _sc_body(idx_hbm, table_hbm, out_hbm,
             idx0, idx1, rows0, rows1, gsem0, gsem1, wsem0, wsem1):
    n_rows = idx_hbm.shape[0]
    rows_per_batch = n_rows // 2
    n_units = n_rows // UNIT
    units_per_batch = rows_per_batch // UNIT

    info = plsc.get_sparse_core_info()
    nc, ns = info.num_cores, info.num_subcores
    n_workers = nc * ns
    wid = lax.axis_index("s") * nc + lax.axis_index("c")

    def load_and_gather(u, idx_v, rows_v, gsem):
        pltpu.sync_copy(idx_hbm.at[pl.ds(u * UNIT, UNIT)], idx_v)
        hs = []
        for k in range(N_FULL):
            hs.append(pltpu.async_copy(
                table_hbm.at[idx_v.at[pl.ds(k * GSUB, GSUB)]],
                rows_v.at[pl.ds(k * GSUB, GSUB)], gsem))
        hs.append(pltpu.async_copy(
            table_hbm.at[idx_v.at[pl.ds(N_FULL * GSUB, TAIL)]],
            rows_v.at[pl.ds(N_FULL * GSUB, TAIL)], gsem))
        return hs

    def fire_writes(u, rows_v, wsem):
        bb = u // units_per_batch
        pos = (u % units_per_batch) * UNIT
        hs = []
        for k in range(N_HEADS):
            g = 2 * k + bb
            base = g * rows_per_batch + pos
            hs.append(pltpu.async_copy(rows_v, out_hbm.at[pl.ds(base, UNIT)], wsem))
        return hs

    u0 = wid
    u1 = wid + n_workers
    u2 = wid + 2 * n_workers

    g0 = load_and_gather(u0, idx0, rows0, gsem0)
    g1 = load_and_gather(u1, idx1, rows1, gsem1)
    for h in g0:
        h.wait()
    w0 = fire_writes(u0, rows0, wsem0)
    for h in g1:
        h.wait()
    w1 = fire_writes(u1, rows1, wsem1)
    for h in w0:
        h.wait()

    @pl.when(u2 < n_units)
    def _():
        g2 = load_and_gather(u2, idx0, rows0, gsem0)
        for h in g2:
            h.wait()
        for h in fire_writes(u2, rows0, wsem0):
            h.wait()

    for h in w1:
        h.wait()


def kernel(src_seq, structure_emb):
    batch, num_posts, _ = src_seq.shape
    idx = src_seq.reshape(-1).astype(jnp.int32)
    table = structure_emb.astype(jnp.float32)
    n_rows_out = batch * N_HEADS * num_posts * num_posts

    mesh = plsc.VectorSubcoreMesh(core_axis_name="c", subcore_axis_name="s")
    f = pl.kernel(
        _sc_body,
        out_type=jax.ShapeDtypeStruct((n_rows_out, D_EMB), jnp.float32),
        mesh=mesh,
        scratch_types=[
            pltpu.VMEM((UNIT,), jnp.int32),
            pltpu.VMEM((UNIT,), jnp.int32),
            pltpu.VMEM((UNIT, D_EMB), jnp.float32),
            pltpu.VMEM((UNIT, D_EMB), jnp.float32),
            pltpu.SemaphoreType.DMA,
            pltpu.SemaphoreType.DMA,
            pltpu.SemaphoreType.DMA,
            pltpu.SemaphoreType.DMA,
        ],
        compiler_params=pltpu.CompilerParams(use_tc_tiling_on_sc=False),
    )
    out = f(idx, table)
    return out.reshape(batch, N_HEADS, num_posts, num_posts, D_EMB)


# speedup vs baseline: 12.3866x; 4.4784x over previous
import jax
import jax.numpy as jnp
from jax import lax
from jax.experimental import pallas as pl
from jax.experimental.pallas import tpu as pltpu
from jax.experimental.pallas import tpu_sc as plsc

N_HEADS = 12
D_EMB = 64
P = 200
N_BATCH = 2
LANES = 16


def _sc_body(idx_hbm, tab_hbm, out_hbm, idx0, idx1, tabv, tr0, tr1, wsem0, wsem1):
    info = plsc.get_sparse_core_info()
    nc, ns = info.num_cores, info.num_subcores
    n_workers = nc * ns
    wid = lax.axis_index("s") * nc + lax.axis_index("c")

    n_tasks = N_BATCH * P
    full_k = n_tasks // n_workers
    extra = n_tasks % n_workers

    pltpu.sync_copy(tab_hbm, tabv)

    jstarts = [jg * LANES for jg in range(P // LANES)] + [P - LANES]

    def build_block(t, idx_v, tr_v):
        bb = t // P
        i = t % P
        pltpu.sync_copy(idx_hbm.at[pl.ds(t * P, P)], idx_v)
        for j0 in jstarts:
            w = idx_v[pl.ds(j0, LANES)] * D_EMB

            def d_body(d, _):
                tr_v[d, pl.ds(j0, LANES)] = plsc.load_gather(tabv, [w + d])
                return 0

            lax.fori_loop(0, D_EMB, d_body, 0)
        return bb, i

    def fire_writes(bb, i, tr_v, wsem):
        hs = []
        for k in range(N_HEADS):
            g = 2 * k + bb
            b_out = g // N_HEADS
            h_out = g % N_HEADS
            hs.append(pltpu.async_copy(tr_v, out_hbm.at[b_out, h_out, i], wsem))
        return hs

    bufs = ((idx0, tr0, wsem0), (idx1, tr1, wsem1))
    pending = [None, None]
    for k in range(full_k):
        idx_v, tr_v, wsem = bufs[k % 2]
        if pending[k % 2] is not None:
            for h in pending[k % 2]:
                h.wait()
        t = wid + k * n_workers
        bb, i = build_block(t, idx_v, tr_v)
        pending[k % 2] = fire_writes(bb, i, tr_v, wsem)

    @pl.when(wid < extra)
    def _():
        idx_v, tr_v, wsem = bufs[full_k % 2]
        for h in pending[full_k % 2]:
            h.wait()
        t = wid + full_k * n_workers
        bb, i = build_block(t, idx_v, tr_v)
        for h in fire_writes(bb, i, tr_v, wsem):
            h.wait()

    @pl.when(wid >= extra)
    def _():
        for h in pending[full_k % 2]:
            h.wait()

    for h in pending[(full_k + 1) % 2]:
        h.wait()


def kernel(src_seq, structure_emb):
    batch, num_posts, _ = src_seq.shape
    idx = src_seq.reshape(-1).astype(jnp.int32)
    tab_flat = structure_emb.astype(jnp.float32).reshape(-1)

    mesh = plsc.VectorSubcoreMesh(core_axis_name="c", subcore_axis_name="s")
    f = pl.kernel(
        _sc_body,
        out_type=jax.ShapeDtypeStruct((batch, N_HEADS, num_posts, D_EMB, num_posts),
                                      jnp.float32),
        mesh=mesh,
        scratch_types=[
            pltpu.VMEM((P,), jnp.int32),
            pltpu.VMEM((P,), jnp.int32),
            pltpu.VMEM((tab_flat.shape[0],), jnp.float32),
            pltpu.VMEM((D_EMB, P), jnp.float32),
            pltpu.VMEM((D_EMB, P), jnp.float32),
            pltpu.SemaphoreType.DMA,
            pltpu.SemaphoreType.DMA,
        ],
        compiler_params=pltpu.CompilerParams(use_tc_tiling_on_sc=True,
                                             needs_layout_passes=False),
    )
    out = f(idx, tab_flat)
    return out.transpose(0, 1, 2, 4, 3)


# speedup vs baseline: 13.5171x; 1.0913x over previous
import jax
import jax.numpy as jnp
from jax import lax
from jax.experimental import pallas as pl
from jax.experimental.pallas import tpu as pltpu
from jax.experimental.pallas import tpu_sc as plsc

N_HEADS = 12
D_EMB = 64
P = 200
N_BATCH = 2
LANES = 16


def _sc_body(idx_hbm, tab_hbm, out_hbm, idx_all, tabv, tr0, tr1,
             isem, wsem0, wsem1):
    info = plsc.get_sparse_core_info()
    nc, ns = info.num_cores, info.num_subcores
    n_workers = nc * ns
    wid = lax.axis_index("s") * nc + lax.axis_index("c")

    n_tasks = N_BATCH * P
    full_k = n_tasks // n_workers
    extra = n_tasks % n_workers
    max_k = full_k + 1

    pltpu.sync_copy(tab_hbm, tabv)

    idx_hs = []
    for k in range(max_k):
        t = wid + k * n_workers
        t = jnp.minimum(t, n_tasks - 1)
        idx_hs.append(pltpu.async_copy(
            idx_hbm.at[pl.ds(t * P, P)], idx_all.at[pl.ds(k * P, P)], isem))

    n_jg = P // LANES + 1

    def build_block(t, k, tr_v):
        bb = t // P
        i = t % P
        idx_hs[k].wait()

        def jg_body(jg, _):
            j0 = jnp.minimum(jg * LANES, P - LANES)
            w = idx_all[pl.ds(k * P + j0, LANES)] * D_EMB

            def d_body(dq, _):
                d = dq * 4
                for c in range(4):
                    tr_v[d + c, pl.ds(j0, LANES)] = plsc.load_gather(tabv, [w + (d + c)])
                return 0

            lax.fori_loop(0, D_EMB // 4, d_body, 0)
            return 0

        lax.fori_loop(0, n_jg, jg_body, 0)
        return bb, i

    def fire_writes(bb, i, tr_v, wsem):
        hs = []
        for k in range(N_HEADS):
            g = 2 * k + bb
            b_out = g // N_HEADS
            h_out = g % N_HEADS
            hs.append(pltpu.async_copy(tr_v, out_hbm.at[b_out, h_out, i], wsem))
        return hs

    bufs = ((tr0, wsem0), (tr1, wsem1))
    pending = [None, None]
    for k in range(full_k):
        tr_v, wsem = bufs[k % 2]
        if pending[k % 2] is not None:
            for h in pending[k % 2]:
                h.wait()
        t = wid + k * n_workers
        bb, i = build_block(t, k, tr_v)
        pending[k % 2] = fire_writes(bb, i, tr_v, wsem)

    @pl.when(wid < extra)
    def _():
        tr_v, wsem = bufs[full_k % 2]
        for h in pending[full_k % 2]:
            h.wait()
        t = wid + full_k * n_workers
        bb, i = build_block(t, full_k, tr_v)
        for h in fire_writes(bb, i, tr_v, wsem):
            h.wait()

    @pl.when(wid >= extra)
    def _():
        for h in pending[full_k % 2]:
            h.wait()
        idx_hs[full_k].wait()

    for h in pending[(full_k + 1) % 2]:
        h.wait()


def kernel(src_seq, structure_emb):
    batch, num_posts, _ = src_seq.shape
    idx = src_seq.reshape(-1).astype(jnp.int32)
    tab_flat = structure_emb.astype(jnp.float32).reshape(-1)

    mesh = plsc.VectorSubcoreMesh(core_axis_name="c", subcore_axis_name="s")
    f = pl.kernel(
        _sc_body,
        out_type=jax.ShapeDtypeStruct((batch, N_HEADS, num_posts, D_EMB, num_posts),
                                      jnp.float32),
        mesh=mesh,
        scratch_types=[
            pltpu.VMEM(((batch * num_posts // 32 + 1) * num_posts,), jnp.int32),
            pltpu.VMEM((tab_flat.shape[0],), jnp.float32),
            pltpu.VMEM((D_EMB, P), jnp.float32),
            pltpu.VMEM((D_EMB, P), jnp.float32),
            pltpu.SemaphoreType.DMA,
            pltpu.SemaphoreType.DMA,
            pltpu.SemaphoreType.DMA,
        ],
        compiler_params=pltpu.CompilerParams(use_tc_tiling_on_sc=True,
                                             needs_layout_passes=False),
    )
    out = f(idx, tab_flat)
    return out.transpose(0, 1, 2, 4, 3)
